# baseline (speedup 1.0000x reference)
import jax
import jax.numpy as jnp
from jax import lax
from jax.experimental import pallas as pl
from jax.experimental.pallas import tpu as pltpu
from jax.experimental.pallas import tpu_sc as plsc

N_NODES = 10000
N_EDGES = 160000
IN_F = 8
OUT_F = 8
D_EDGE = 16
CH = IN_F * OUT_F
NW = 32
NPT = 313
LAST_CNT = N_NODES - (NW - 1) * NPT
CHUNK = 2000
NCHUNK = N_EDGES // CHUNK
CAP = 11008
BB = 64
NEG = -3.0e38


def _mm_body(ef, w, b, o):
    o[...] = jnp.dot(ef[...], w[...],
                     preferred_element_type=jnp.float32) + b[...]


def _edge_mats(efeat, W_cat, b_cat):
    BLK = 2000
    return pl.pallas_call(
        _mm_body,
        grid=(N_EDGES // BLK,),
        in_specs=[
            pl.BlockSpec((BLK, D_EDGE), lambda i: (i, 0)),
            pl.BlockSpec((D_EDGE, 2 * CH), lambda i: (0, 0)),
            pl.BlockSpec((1, 2 * CH), lambda i: (0, 0)),
        ],
        out_specs=pl.BlockSpec((BLK, 2 * CH), lambda i: (i, 0)),
        out_shape=jax.ShapeDtypeStruct((N_EDGES, 2 * CH), jnp.float32),
    )(efeat, W_cat, b_cat.reshape(1, 2 * CH))


def _sc_body(wawm_hbm, featp_hbm, src_hbm, dst_hbm, out_hbm,
             amax0, amax1, amax2, amax3,
             aden0, aden1, aden2, aden3,
             anum0, anum1, anum2, anum3,
             outb, dstcA, srccA, dstcB, srccB,
             ce, cs, cd,
             wbA, hA, wbB, hB,
             sem_cA, sem_cB, sem_aA, sem_hA, sem_aB, sem_hB):
    amax = [amax0, amax1, amax2, amax3]
    aden = [aden0, aden1, aden2, aden3]
    anum = [anum0, anum1, anum2, anum3]
    wid = lax.axis_index("s") * 2 + lax.axis_index("c")
    lo = wid * NPT
    cnt = jnp.minimum(N_NODES - lo, NPT)
    iota = lax.iota(jnp.int32, 16)
    lane8 = (iota >= 8).astype(jnp.int32)
    io7 = iota & 7
    negv = jnp.full((16,), NEG, jnp.float32)
    zerov = jnp.zeros((16,), jnp.float32)
    padv = jnp.full((16,), NPT, jnp.int32)
    zeroiv = jnp.zeros((16,), jnp.int32)

    def init_acc(i, _):
        s = pl.ds(i * 16, 16)
        for k in range(4):
            amax[k][s] = negv
            aden[k][s] = zerov
            anum[k][s] = zerov
        return 0

    lax.fori_loop(0, NPT + 1, init_acc, 0)

    def init_out(i, _):
        outb[pl.ds(i * 16, 16)] = zerov
        return 0

    lax.fori_loop(0, (NPT * OUT_F + 32) // 16, init_out, 0)

    def fire_chunk(ci, dbuf, sbuf, sem):
        base = ci * CHUNK
        pltpu.async_copy(dst_hbm.at[pl.ds(base, CHUNK)], dbuf, sem)
        pltpu.async_copy(src_hbm.at[pl.ds(base, CHUNK)], sbuf, sem)

    def drain_chunk(dbuf, sbuf, sem):
        pltpu.make_async_copy(dst_hbm.at[pl.ds(0, CHUNK)], dbuf, sem).wait()
        pltpu.make_async_copy(src_hbm.at[pl.ds(0, CHUNK)], sbuf, sem).wait()

    def scan_chunk(ci, off, dbuf, sbuf):
        base = ci * CHUNK

        def bin_body(v, offv):
            d = dbuf[pl.ds(v * 16, 16)]
            m = (d >= lo) & (d < lo + cnt)
            cum = plsc.cumsum(m.astype(jnp.int32))
            pos = offv + cum - 1
            posc = jnp.where(pos < CAP, pos, CAP + BB + iota)
            plsc.store_scatter(cd, [posc], d - lo, mask=m)
            plsc.store_scatter(cs, [posc], sbuf[pl.ds(v * 16, 16)], mask=m)
            plsc.store_scatter(ce, [posc], iota + (base + v * 16), mask=m)
            return offv + plsc.all_reduce_population_count(m)

        return lax.fori_loop(0, CHUNK // 16, bin_body, off, unroll=5)

    fire_chunk(0, dstcA, srccA, sem_cA)

    def chunk_loop(ci, off):
        even = (ci % 2) == 0

        def do(dbuf, sbuf, sem, ndbuf, nsbuf, nsem):
            @pl.when(ci + 1 < NCHUNK)
            def _():
                fire_chunk(ci + 1, ndbuf, nsbuf, nsem)
            drain_chunk(dbuf, sbuf, sem)
            return scan_chunk(ci, off, dbuf, sbuf)

        return lax.cond(
            even,
            lambda: do(dstcA, srccA, sem_cA, dstcB, srccB, sem_cB),
            lambda: do(dstcB, srccB, sem_cB, dstcA, srccA, sem_cA),
        )

    totalv = lax.fori_loop(0, NCHUNK, chunk_loop, jnp.zeros((16,), jnp.int32))
    total = totalv[0]
    tcap = jnp.minimum(total, CAP)
    for t in range(4):
        s = pl.ds(tcap + 16 * t, 16)
        cd[s] = padv
        cs[s] = zeroiv
        ce[s] = zeroiv

    def fire_batch(t, wbuf, hbuf, sema, semh):
        eb = ce.at[pl.ds(t * BB, BB)]
        sb = cs.at[pl.ds(t * BB, BB)]
        pltpu.async_copy(wawm_hbm.at[eb], wbuf, sema)
        pltpu.async_copy(featp_hbm.at[sb], hbuf, semh)

    def drain_batch(wbuf, hbuf, sema, semh):
        eb0 = ce.at[pl.ds(0, BB)]
        sb0 = cs.at[pl.ds(0, BB)]
        pltpu.make_async_copy(wawm_hbm.at[eb0], wbuf, sema).wait()
        pltpu.make_async_copy(featp_hbm.at[sb0], hbuf, semh).wait()

    def proc_batch(t, wbuf, hbuf):
        def edge(j, _):
            dl = plsc.load_gather(cd, [jnp.full((16,), t * BB + j, jnp.int32)])
            idx = dl * 16 + iota
            js = jnp.full((16,), j, jnp.int32)
            for k in range(4):
                hk = plsc.load_gather(hbuf, [js, lane8 + 2 * k])
                wak = plsc.load_gather(wbuf, [js, iota + 16 * k])
                wmk = plsc.load_gather(wbuf, [js, CH + iota + 16 * k])
                e2 = wak * hk
                mold = plsc.load_gather(amax[k], [idx])
                mnew = jnp.maximum(mold, e2)
                alpha = jnp.exp(mold - mnew)
                p = jnp.exp(e2 - mnew)
                dold = plsc.load_gather(aden[k], [idx])
                nold = plsc.load_gather(anum[k], [idx])
                plsc.store_scatter(amax[k], [idx], mnew)
                plsc.store_scatter(aden[k], [idx], dold * alpha + p)
                plsc.store_scatter(anum[k], [idx], nold * alpha + wmk * hk * p)
            return 0

        lax.fori_loop(0, BB, edge, 0, unroll=8)

    def cached_sweep():
        nb = (tcap + BB - 1) // BB

        @pl.when(nb > 0)
        def _():
            fire_batch(0, wbA, hA, sem_aA, sem_hA)

        def body(t, _):
            def do(wb, hb, sa, sh, nwb, nhb, nsa, nsh):
                @pl.when(t + 1 < nb)
                def _():
                    fire_batch(t + 1, nwb, nhb, nsa, nsh)
                drain_batch(wb, hb, sa, sh)
                proc_batch(t, wb, hb)
                return 0

            return lax.cond(
                (t % 2) == 0,
                lambda: do(wbA, hA, sem_aA, sem_hA, wbB, hB, sem_aB, sem_hB),
                lambda: do(wbB, hB, sem_aB, sem_hB, wbA, hA, sem_aA, sem_hA),
            )

        lax.fori_loop(0, nb, body, 0)

    def fb_fetch(b):
        eb = ce.at[pl.ds(b * 16, 16)]
        sb = cs.at[pl.ds(b * 16, 16)]
        wb16 = wbA.at[pl.ds(0, 16)]
        h16 = hA.at[pl.ds(0, 16)]
        cpa = pltpu.async_copy(wawm_hbm.at[eb], wb16, sem_aA)
        cph = pltpu.async_copy(featp_hbm.at[sb], h16, sem_hA)
        cpa.wait()
        cph.wait()

    def fb_mb(b, _):
        fb_fetch(b)

        def edge(j, _):
            dl = plsc.load_gather(cd, [jnp.full((16,), b * 16 + j, jnp.int32)])
            idx = dl * 16 + iota
            js = jnp.full((16,), j, jnp.int32)
            for k in range(4):
                hk = plsc.load_gather(hA, [js, lane8 + 2 * k])
                wak = plsc.load_gather(wbA, [js, iota + 16 * k])
                wmk = plsc.load_gather(wbA, [js, CH + iota + 16 * k])
                e2 = wak * hk
                mold = plsc.load_gather(amax[k], [idx])
                mnew = jnp.maximum(mold, e2)
                alpha = jnp.exp(mold - mnew)
                p = jnp.exp(e2 - mnew)
                dold = plsc.load_gather(aden[k], [idx])
                nold = plsc.load_gather(anum[k], [idx])
                plsc.store_scatter(amax[k], [idx], mnew)
                plsc.store_scatter(aden[k], [idx], dold * alpha + p)
                plsc.store_scatter(anum[k], [idx], nold * alpha + wmk * hk * p)
            return 0

        lax.fori_loop(0, 16, edge, 0, unroll=8)
        return 0

    def fb_sweep():
        def chunk_body(ci, _):
            base = ci * CHUNK
            pltpu.sync_copy(dst_hbm.at[pl.ds(base, CHUNK)], dstcA)
            pltpu.sync_copy(src_hbm.at[pl.ds(base, CHUNK)], srccA)

            def bin_body(v, off):
                d = dstcA[pl.ds(v * 16, 16)]
                m = (d >= lo) & (d < lo + cnt)
                cum = plsc.cumsum(m.astype(jnp.int32))
                pos = off + cum - 1
                plsc.store_scatter(cd, [pos], d - lo, mask=m)
                plsc.store_scatter(cs, [pos], srccA[pl.ds(v * 16, 16)], mask=m)
                plsc.store_scatter(ce, [pos], iota + (base + v * 16), mask=m)
                return off + cum[15]

            n = lax.fori_loop(0, CHUNK // 16, bin_body, jnp.int32(0), unroll=5)
            tl = pl.ds(n, 16)
            cd[tl] = padv
            cs[tl] = zeroiv
            ce[tl] = zeroiv
            lax.fori_loop(0, (n + 15) // 16, fb_mb, 0)
            return 0

        lax.fori_loop(0, NCHUNK, chunk_body, 0)

    ok = total <= CAP

    @pl.when(ok)
    def _():
        cached_sweep()

    @pl.when(jnp.logical_not(ok))
    def _():
        fb_sweep()

    def fin(nn, _):
        acc = zerov
        for k in range(4):
            s = pl.ds(nn * 16, 16)
            acc = acc + anum[k][s] / jnp.maximum(aden[k][s], 1.0)
        oidx = nn * OUT_F + io7
        plsc.addupdate_scatter(outb, [oidx], acc, mask=iota < 8)
        plsc.addupdate_scatter(outb, [oidx], acc, mask=iota >= 8)
        return 0

    lax.fori_loop(0, cnt, fin, 0)

    @pl.when(wid < NW - 1)
    def _():
        pltpu.sync_copy(outb.at[pl.ds(0, NPT * OUT_F)],
                        out_hbm.at[pl.ds(lo * OUT_F, NPT * OUT_F)])

    @pl.when(wid == NW - 1)
    def _():
        pltpu.sync_copy(outb.at[pl.ds(0, LAST_CNT * OUT_F)],
                        out_hbm.at[pl.ds(lo * OUT_F, LAST_CNT * OUT_F)])


def _sc_call(wawm, featp, src, dst):
    kern = pl.kernel(
        _sc_body,
        out_type=jax.ShapeDtypeStruct((N_NODES * OUT_F,), jnp.float32),
        mesh=plsc.VectorSubcoreMesh(core_axis_name="c", subcore_axis_name="s",
                                    num_cores=2, num_subcores=16),
        scratch_types=[
            pltpu.VMEM(((NPT + 1) * 16,), jnp.float32),
            pltpu.VMEM(((NPT + 1) * 16,), jnp.float32),
            pltpu.VMEM(((NPT + 1) * 16,), jnp.float32),
            pltpu.VMEM(((NPT + 1) * 16,), jnp.float32),
            pltpu.VMEM(((NPT + 1) * 16,), jnp.float32),
            pltpu.VMEM(((NPT + 1) * 16,), jnp.float32),
            pltpu.VMEM(((NPT + 1) * 16,), jnp.float32),
            pltpu.VMEM(((NPT + 1) * 16,), jnp.float32),
            pltpu.VMEM(((NPT + 1) * 16,), jnp.float32),
            pltpu.VMEM(((NPT + 1) * 16,), jnp.float32),
            pltpu.VMEM(((NPT + 1) * 16,), jnp.float32),
            pltpu.VMEM(((NPT + 1) * 16,), jnp.float32),
            pltpu.VMEM((NPT * OUT_F + 32,), jnp.float32),
            pltpu.VMEM((CHUNK,), jnp.int32),
            pltpu.VMEM((CHUNK,), jnp.int32),
            pltpu.VMEM((CHUNK,), jnp.int32),
            pltpu.VMEM((CHUNK,), jnp.int32),
            pltpu.VMEM((CAP + 80,), jnp.int32),
            pltpu.VMEM((CAP + 80,), jnp.int32),
            pltpu.VMEM((CAP + 80,), jnp.int32),
            pltpu.VMEM((BB, 2 * CH), jnp.float32),
            pltpu.VMEM((BB, D_EDGE), jnp.float32),
            pltpu.VMEM((BB, 2 * CH), jnp.float32),
            pltpu.VMEM((BB, D_EDGE), jnp.float32),
            pltpu.SemaphoreType.DMA,
            pltpu.SemaphoreType.DMA,
            pltpu.SemaphoreType.DMA,
            pltpu.SemaphoreType.DMA,
            pltpu.SemaphoreType.DMA,
            pltpu.SemaphoreType.DMA,
        ],
        compiler_params=pltpu.CompilerParams(
            needs_layout_passes=False, use_tc_tiling_on_sc=False),
    )
    return kern(wawm, featp, src, dst)


def kernel(feat, efeat, W_msg, b_msg, W_attn, b_attn, edge_index):
    W_cat = jnp.concatenate([W_attn, W_msg], axis=1)
    b_cat = jnp.concatenate([b_attn, b_msg], axis=0)
    wawm = _edge_mats(efeat, W_cat, b_cat)
    featp = jnp.pad(feat, ((0, 0), (0, D_EDGE - IN_F)))
    src = edge_index[0].astype(jnp.int32)
    dst = edge_index[1].astype(jnp.int32)
    out = _sc_call(wawm, featp, src, dst)
    return out.reshape(N_NODES, OUT_F)


# speedup vs baseline: 1.0073x; 1.0073x over previous
import jax
import jax.numpy as jnp
from jax import lax
from jax.experimental import pallas as pl
from jax.experimental.pallas import tpu as pltpu
from jax.experimental.pallas import tpu_sc as plsc

N_NODES = 10000
N_EDGES = 160000
IN_F = 8
OUT_F = 8
D_EDGE = 16
CH = IN_F * OUT_F
NW = 32
NPT = 313
LAST_CNT = N_NODES - (NW - 1) * NPT
CHUNK = 2000
NCHUNK = N_EDGES // CHUNK
CAP = 11008
BB = 64
NEG = -3.0e38


def _mm_body(ef, w, b, o):
    o[...] = jnp.dot(ef[...], w[...],
                     preferred_element_type=jnp.float32) + b[...]


def _edge_mats(efeat, W_cat, b_cat):
    BLK = 2000
    return pl.pallas_call(
        _mm_body,
        grid=(N_EDGES // BLK,),
        in_specs=[
            pl.BlockSpec((BLK, D_EDGE), lambda i: (i, 0)),
            pl.BlockSpec((D_EDGE, 2 * CH), lambda i: (0, 0)),
            pl.BlockSpec((1, 2 * CH), lambda i: (0, 0)),
        ],
        out_specs=pl.BlockSpec((BLK, 2 * CH), lambda i: (i, 0)),
        out_shape=jax.ShapeDtypeStruct((N_EDGES, 2 * CH), jnp.float32),
    )(efeat, W_cat, b_cat.reshape(1, 2 * CH))


def _sc_body(wawm_hbm, featx_hbm, src_hbm, dst_hbm, out_hbm,
             amax0, amax1, amax2, amax3,
             aden0, aden1, aden2, aden3,
             anum0, anum1, anum2, anum3,
             outb, dstcA, srccA, dstcB, srccB,
             ce, cs, cd,
             wbA, hA, wbB, hB,
             sem_cA, sem_cB, sem_aA, sem_hA, sem_aB, sem_hB):
    amax = [amax0, amax1, amax2, amax3]
    aden = [aden0, aden1, aden2, aden3]
    anum = [anum0, anum1, anum2, anum3]
    wid = lax.axis_index("s") * 2 + lax.axis_index("c")
    lo = wid * NPT
    cnt = jnp.minimum(N_NODES - lo, NPT)
    iota = lax.iota(jnp.int32, 16)
    lane8 = (iota >= 8).astype(jnp.int32)
    io7 = iota & 7
    negv = jnp.full((16,), NEG, jnp.float32)
    zerov = jnp.zeros((16,), jnp.float32)
    padv = jnp.full((16,), NPT, jnp.int32)
    zeroiv = jnp.zeros((16,), jnp.int32)

    def init_acc(i, _):
        s = pl.ds(i * 16, 16)
        for k in range(4):
            amax[k][s] = negv
            aden[k][s] = zerov
            anum[k][s] = zerov
        return 0

    lax.fori_loop(0, NPT + 1, init_acc, 0)

    def init_out(i, _):
        outb[pl.ds(i * 16, 16)] = zerov
        return 0

    lax.fori_loop(0, (NPT * OUT_F + 32) // 16, init_out, 0)

    def fire_chunk(ci, dbuf, sbuf, sem):
        base = ci * CHUNK
        pltpu.async_copy(dst_hbm.at[pl.ds(base, CHUNK)], dbuf, sem)
        pltpu.async_copy(src_hbm.at[pl.ds(base, CHUNK)], sbuf, sem)

    def drain_chunk(dbuf, sbuf, sem):
        pltpu.make_async_copy(dst_hbm.at[pl.ds(0, CHUNK)], dbuf, sem).wait()
        pltpu.make_async_copy(src_hbm.at[pl.ds(0, CHUNK)], sbuf, sem).wait()

    def scan_chunk(ci, off, dbuf, sbuf):
        base = ci * CHUNK

        def bin_body(v, offv):
            d = dbuf[pl.ds(v * 16, 16)]
            m = (d >= lo) & (d < lo + cnt)
            cum = plsc.cumsum(m.astype(jnp.int32))
            pos = offv + cum - 1
            posc = jnp.where(pos < CAP, pos, CAP + BB + iota)
            plsc.store_scatter(cd, [posc], d - lo, mask=m)
            plsc.store_scatter(cs, [posc], sbuf[pl.ds(v * 16, 16)], mask=m)
            plsc.store_scatter(ce, [posc], iota + (base + v * 16), mask=m)
            return offv + plsc.all_reduce_population_count(m)

        return lax.fori_loop(0, CHUNK // 16, bin_body, off, unroll=5)

    fire_chunk(0, dstcA, srccA, sem_cA)

    def chunk_loop(ci, off):
        even = (ci % 2) == 0

        def do(dbuf, sbuf, sem, ndbuf, nsbuf, nsem):
            @pl.when(ci + 1 < NCHUNK)
            def _():
                fire_chunk(ci + 1, ndbuf, nsbuf, nsem)
            drain_chunk(dbuf, sbuf, sem)
            return scan_chunk(ci, off, dbuf, sbuf)

        return lax.cond(
            even,
            lambda: do(dstcA, srccA, sem_cA, dstcB, srccB, sem_cB),
            lambda: do(dstcB, srccB, sem_cB, dstcA, srccA, sem_cA),
        )

    totalv = lax.fori_loop(0, NCHUNK, chunk_loop, jnp.zeros((16,), jnp.int32))
    total = totalv[0]
    tcap = jnp.minimum(total, CAP)
    for t in range(4):
        s = pl.ds(tcap + 16 * t, 16)
        cd[s] = padv
        cs[s] = zeroiv
        ce[s] = zeroiv

    def fire_batch(t, wbuf, hbuf, sema, semh):
        eb = ce.at[pl.ds(t * BB, BB)]
        sb = cs.at[pl.ds(t * BB, BB)]
        pltpu.async_copy(wawm_hbm.at[eb], wbuf, sema)
        pltpu.async_copy(featx_hbm.at[sb], hbuf, semh)

    def drain_batch(wbuf, hbuf, sema, semh):
        eb0 = ce.at[pl.ds(0, BB)]
        sb0 = cs.at[pl.ds(0, BB)]
        pltpu.make_async_copy(wawm_hbm.at[eb0], wbuf, sema).wait()
        pltpu.make_async_copy(featx_hbm.at[sb0], hbuf, semh).wait()

    def proc_batch(t, wbuf, hbuf):
        def edge(j, _):
            dl = plsc.load_gather(cd, [jnp.full((16,), t * BB + j, jnp.int32)])
            idx = dl * 16 + iota
            for k in range(4):
                hk = hbuf[j, pl.ds(16 * k, 16)]
                wak = wbuf[j, pl.ds(16 * k, 16)]
                wmk = wbuf[j, pl.ds(CH + 16 * k, 16)]
                e2 = wak * hk
                mold = plsc.load_gather(amax[k], [idx])
                mnew = jnp.maximum(mold, e2)
                alpha = jnp.exp(mold - mnew)
                p = jnp.exp(e2 - mnew)
                dold = plsc.load_gather(aden[k], [idx])
                nold = plsc.load_gather(anum[k], [idx])
                plsc.store_scatter(amax[k], [idx], mnew)
                plsc.store_scatter(aden[k], [idx], dold * alpha + p)
                plsc.store_scatter(anum[k], [idx], nold * alpha + wmk * hk * p)
            return 0

        lax.fori_loop(0, BB, edge, 0, unroll=8)

    def cached_sweep():
        nb = (tcap + BB - 1) // BB

        @pl.when(nb > 0)
        def _():
            fire_batch(0, wbA, hA, sem_aA, sem_hA)

        def body(t, _):
            def do(wb, hb, sa, sh, nwb, nhb, nsa, nsh):
                @pl.when(t + 1 < nb)
                def _():
                    fire_batch(t + 1, nwb, nhb, nsa, nsh)
                drain_batch(wb, hb, sa, sh)
                proc_batch(t, wb, hb)
                return 0

            return lax.cond(
                (t % 2) == 0,
                lambda: do(wbA, hA, sem_aA, sem_hA, wbB, hB, sem_aB, sem_hB),
                lambda: do(wbB, hB, sem_aB, sem_hB, wbA, hA, sem_aA, sem_hA),
            )

        lax.fori_loop(0, nb, body, 0)

    def fb_fetch(b):
        eb = ce.at[pl.ds(b * 16, 16)]
        sb = cs.at[pl.ds(b * 16, 16)]
        wb16 = wbA.at[pl.ds(0, 16)]
        h16 = hA.at[pl.ds(0, 16)]
        cpa = pltpu.async_copy(wawm_hbm.at[eb], wb16, sem_aA)
        cph = pltpu.async_copy(featx_hbm.at[sb], h16, sem_hA)
        cpa.wait()
        cph.wait()

    def fb_mb(b, _):
        fb_fetch(b)

        def edge(j, _):
            dl = plsc.load_gather(cd, [jnp.full((16,), b * 16 + j, jnp.int32)])
            idx = dl * 16 + iota
            for k in range(4):
                hk = hA[j, pl.ds(16 * k, 16)]
                wak = wbA[j, pl.ds(16 * k, 16)]
                wmk = wbA[j, pl.ds(CH + 16 * k, 16)]
                e2 = wak * hk
                mold = plsc.load_gather(amax[k], [idx])
                mnew = jnp.maximum(mold, e2)
                alpha = jnp.exp(mold - mnew)
                p = jnp.exp(e2 - mnew)
                dold = plsc.load_gather(aden[k], [idx])
                nold = plsc.load_gather(anum[k], [idx])
                plsc.store_scatter(amax[k], [idx], mnew)
                plsc.store_scatter(aden[k], [idx], dold * alpha + p)
                plsc.store_scatter(anum[k], [idx], nold * alpha + wmk * hk * p)
            return 0

        lax.fori_loop(0, 16, edge, 0, unroll=8)
        return 0

    def fb_sweep():
        def chunk_body(ci, _):
            base = ci * CHUNK
            pltpu.sync_copy(dst_hbm.at[pl.ds(base, CHUNK)], dstcA)
            pltpu.sync_copy(src_hbm.at[pl.ds(base, CHUNK)], srccA)

            def bin_body(v, off):
                d = dstcA[pl.ds(v * 16, 16)]
                m = (d >= lo) & (d < lo + cnt)
                cum = plsc.cumsum(m.astype(jnp.int32))
                pos = off + cum - 1
                plsc.store_scatter(cd, [pos], d - lo, mask=m)
                plsc.store_scatter(cs, [pos], srccA[pl.ds(v * 16, 16)], mask=m)
                plsc.store_scatter(ce, [pos], iota + (base + v * 16), mask=m)
                return off + cum[15]

            n = lax.fori_loop(0, CHUNK // 16, bin_body, jnp.int32(0), unroll=5)
            tl = pl.ds(n, 16)
            cd[tl] = padv
            cs[tl] = zeroiv
            ce[tl] = zeroiv
            lax.fori_loop(0, (n + 15) // 16, fb_mb, 0)
            return 0

        lax.fori_loop(0, NCHUNK, chunk_body, 0)

    ok = total <= CAP

    @pl.when(ok)
    def _():
        cached_sweep()

    @pl.when(jnp.logical_not(ok))
    def _():
        fb_sweep()

    def fin(nn, _):
        acc = zerov
        for k in range(4):
            s = pl.ds(nn * 16, 16)
            acc = acc + anum[k][s] / jnp.maximum(aden[k][s], 1.0)
        oidx = nn * OUT_F + io7
        plsc.addupdate_scatter(outb, [oidx], acc, mask=iota < 8)
        plsc.addupdate_scatter(outb, [oidx], acc, mask=iota >= 8)
        return 0

    lax.fori_loop(0, cnt, fin, 0)

    @pl.when(wid < NW - 1)
    def _():
        pltpu.sync_copy(outb.at[pl.ds(0, NPT * OUT_F)],
                        out_hbm.at[pl.ds(lo * OUT_F, NPT * OUT_F)])

    @pl.when(wid == NW - 1)
    def _():
        pltpu.sync_copy(outb.at[pl.ds(0, LAST_CNT * OUT_F)],
                        out_hbm.at[pl.ds(lo * OUT_F, LAST_CNT * OUT_F)])


def _sc_call(wawm, featx, src, dst):
    kern = pl.kernel(
        _sc_body,
        out_type=jax.ShapeDtypeStruct((N_NODES * OUT_F,), jnp.float32),
        mesh=plsc.VectorSubcoreMesh(core_axis_name="c", subcore_axis_name="s",
                                    num_cores=2, num_subcores=16),
        scratch_types=[
            pltpu.VMEM(((NPT + 1) * 16,), jnp.float32),
            pltpu.VMEM(((NPT + 1) * 16,), jnp.float32),
            pltpu.VMEM(((NPT + 1) * 16,), jnp.float32),
            pltpu.VMEM(((NPT + 1) * 16,), jnp.float32),
            pltpu.VMEM(((NPT + 1) * 16,), jnp.float32),
            pltpu.VMEM(((NPT + 1) * 16,), jnp.float32),
            pltpu.VMEM(((NPT + 1) * 16,), jnp.float32),
            pltpu.VMEM(((NPT + 1) * 16,), jnp.float32),
            pltpu.VMEM(((NPT + 1) * 16,), jnp.float32),
            pltpu.VMEM(((NPT + 1) * 16,), jnp.float32),
            pltpu.VMEM(((NPT + 1) * 16,), jnp.float32),
            pltpu.VMEM(((NPT + 1) * 16,), jnp.float32),
            pltpu.VMEM((NPT * OUT_F + 32,), jnp.float32),
            pltpu.VMEM((CHUNK,), jnp.int32),
            pltpu.VMEM((CHUNK,), jnp.int32),
            pltpu.VMEM((CHUNK,), jnp.int32),
            pltpu.VMEM((CHUNK,), jnp.int32),
            pltpu.VMEM((CAP + 80,), jnp.int32),
            pltpu.VMEM((CAP + 80,), jnp.int32),
            pltpu.VMEM((CAP + 80,), jnp.int32),
            pltpu.VMEM((BB, 2 * CH), jnp.float32),
            pltpu.VMEM((BB, CH), jnp.float32),
            pltpu.VMEM((BB, 2 * CH), jnp.float32),
            pltpu.VMEM((BB, CH), jnp.float32),
            pltpu.SemaphoreType.DMA,
            pltpu.SemaphoreType.DMA,
            pltpu.SemaphoreType.DMA,
            pltpu.SemaphoreType.DMA,
            pltpu.SemaphoreType.DMA,
            pltpu.SemaphoreType.DMA,
        ],
        compiler_params=pltpu.CompilerParams(
            needs_layout_passes=False, use_tc_tiling_on_sc=False),
    )
    return kern(wawm, featx, src, dst)


def kernel(feat, efeat, W_msg, b_msg, W_attn, b_attn, edge_index):
    W_cat = jnp.concatenate([W_attn, W_msg], axis=1)
    b_cat = jnp.concatenate([b_attn, b_msg], axis=0)
    wawm = _edge_mats(efeat, W_cat, b_cat)
    featx = jnp.repeat(feat, OUT_F, axis=1)
    src = edge_index[0].astype(jnp.int32)
    dst = edge_index[1].astype(jnp.int32)
    out = _sc_call(wawm, featx, src, dst)
    return out.reshape(N_NODES, OUT_F)
